# baseline (speedup 1.0000x reference)
import functools

import jax
import jax.numpy as jnp
from jax import lax
from jax.experimental import pallas as pl
from jax.experimental.pallas import tpu as pltpu
from jax.experimental.pallas import tpu_sc as plsc

L = 16
NB = 4
CS = 4


def kernel(input_tensor, token_table, pos_table):
    B, S = input_tensor.shape
    V, D = token_table.shape
    N = B * S
    NC, NS = 2, 16
    NW = NC * NS
    SPW = S // NW
    NG = SPW // CS
    CR = NB * CS

    idx_flat = input_tensor.reshape(N).astype(jnp.int32)

    mesh = plsc.VectorSubcoreMesh(core_axis_name="c", subcore_axis_name="s")

    @functools.partial(
        pl.kernel,
        mesh=mesh,
        out_type=jax.ShapeDtypeStruct((N, D), jnp.float32),
        scratch_types=[
            pltpu.VMEM((NB, SPW), jnp.int32),
            pltpu.VMEM((NG, CR), jnp.int32),
            pltpu.VMEM((2, CR, D), jnp.float32),
            pltpu.VMEM((2, CS, D), jnp.float32),
            pltpu.SemaphoreType.DMA,
            pltpu.SemaphoreType.DMA,
            pltpu.SemaphoreType.DMA,
            pltpu.SemaphoreType.DMA,
            pltpu.SemaphoreType.DMA,
            pltpu.SemaphoreType.DMA,
        ],
    )
    def emb_kernel(idx_hbm, tok_hbm, pos_hbm, out_hbm,
                   idx_va, idx2, tok_v, pos_v,
                   gsem0, gsem1, psem0, psem1, ssem0, ssem1):
        gsem = (gsem0, gsem1)
        psem = (psem0, psem1)
        ssem = (ssem0, ssem1)
        wid = lax.axis_index("s") * NC + lax.axis_index("c")
        s0 = wid * SPW

        for b in range(NB):
            pltpu.sync_copy(idx_hbm.at[pl.ds(b * S + s0, SPW)], idx_va.at[b])
        for g in range(NG):
            for b in range(NB):
                idx2[g, pl.ds(b * CS, CS)] = idx_va[b, pl.ds(g * CS, CS)]

        def issue(g, p):
            pltpu.async_copy(pos_hbm.at[pl.ds(s0 + g * CS, CS)],
                             pos_v.at[p], psem[p])
            pltpu.async_copy(tok_hbm.at[idx2.at[g]], tok_v.at[p], gsem[p])

        def drain_stores(p):
            for b in range(NB):
                pltpu.make_async_copy(tok_v.at[p, pl.ds(b * CS, CS)],
                                      out_hbm.at[pl.ds(0, CS)],
                                      ssem[p]).wait()

        issue(0, 0)

        def section(g, p):
            pn = 1 - p

            @pl.when(g + 1 < NG)
            def _():
                @pl.when(g >= 1)
                def _():
                    drain_stores(pn)
                issue(g + 1, pn)

            pltpu.make_async_copy(tok_hbm.at[idx2.at[0]],
                                  tok_v.at[p], gsem[p]).wait()
            pltpu.make_async_copy(pos_hbm.at[pl.ds(0, CS)],
                                  pos_v.at[p], psem[p]).wait()

            def row_body(r, carry):
                for d in range(D // L):
                    sl = pl.ds(d * L, L)
                    pv = pos_v[p, r, sl]
                    for b in range(NB):
                        plsc.addupdate(tok_v.at[p, b * CS + r, sl], pv)
                return carry

            lax.fori_loop(0, CS, row_body, 0)

            for b in range(NB):
                pltpu.async_copy(tok_v.at[p, pl.ds(b * CS, CS)],
                                 out_hbm.at[pl.ds(b * S + s0 + g * CS, CS)],
                                 ssem[p])

        def pair_body(gp, carry):
            section(2 * gp, 0)
            section(2 * gp + 1, 1)
            return carry

        lax.fori_loop(0, NG // 2, pair_body, 0)

        drain_stores(0)
        drain_stores(1)

    out = emb_kernel(idx_flat, token_table, pos_table)
    return out.reshape(B, S, D)


# speedup vs baseline: 1.0943x; 1.0943x over previous
import functools

import jax
import jax.numpy as jnp
from jax import lax
from jax.experimental import pallas as pl
from jax.experimental.pallas import tpu as pltpu
from jax.experimental.pallas import tpu_sc as plsc

L = 16
NB = 4
CS = 8
NBUF = 4


def kernel(input_tensor, token_table, pos_table):
    B, S = input_tensor.shape
    V, D = token_table.shape
    N = B * S
    NC, NS = 2, 16
    NW = NC * NS
    SPW = S // NW
    NG = SPW // CS
    CR = NB * CS

    idx_flat = input_tensor.reshape(N).astype(jnp.int32)

    mesh = plsc.VectorSubcoreMesh(core_axis_name="c", subcore_axis_name="s")

    @functools.partial(
        pl.kernel,
        mesh=mesh,
        out_type=jax.ShapeDtypeStruct((N, D), jnp.float32),
        scratch_types=[
            pltpu.VMEM((NB, SPW), jnp.int32),
            pltpu.VMEM((NG, CR), jnp.int32),
            pltpu.VMEM((NBUF, CR, D), jnp.float32),
            pltpu.VMEM((NBUF, CS, D), jnp.float32),
        ] + [pltpu.SemaphoreType.DMA] * (3 * NBUF),
    )
    def emb_kernel(idx_hbm, tok_hbm, pos_hbm, out_hbm,
                   idx_va, idx2, tok_v, pos_v, *sems):
        gsem = sems[0:NBUF]
        psem = sems[NBUF:2 * NBUF]
        ssem = sems[2 * NBUF:3 * NBUF]
        wid = lax.axis_index("s") * NC + lax.axis_index("c")
        s0 = wid * SPW

        for b in range(NB):
            pltpu.sync_copy(idx_hbm.at[pl.ds(b * S + s0, SPW)], idx_va.at[b])
        for g in range(NG):
            for b in range(NB):
                idx2[g, pl.ds(b * CS, CS)] = idx_va[b, pl.ds(g * CS, CS)]

        def issue(g, p):
            pltpu.async_copy(pos_hbm.at[pl.ds(s0 + g * CS, CS)],
                             pos_v.at[p], psem[p])
            pltpu.async_copy(tok_hbm.at[idx2.at[g]], tok_v.at[p], gsem[p])

        def drain_stores(p):
            for b in range(NB):
                pltpu.make_async_copy(tok_v.at[p, pl.ds(b * CS, CS)],
                                      out_hbm.at[pl.ds(0, CS)],
                                      ssem[p]).wait()

        for c in range(NBUF - 1):
            issue(c, c)

        def section(g, p):
            pf = (p + NBUF - 1) % NBUF

            @pl.when(g + NBUF - 1 < NG)
            def _():
                @pl.when(g >= 1)
                def _():
                    drain_stores(pf)
                issue(g + NBUF - 1, pf)

            pltpu.make_async_copy(tok_hbm.at[idx2.at[0]],
                                  tok_v.at[p], gsem[p]).wait()
            pltpu.make_async_copy(pos_hbm.at[pl.ds(0, CS)],
                                  pos_v.at[p], psem[p]).wait()

            def row_body(r, carry):
                for d in range(D // L):
                    sl = pl.ds(d * L, L)
                    pv = pos_v[p, r, sl]
                    for b in range(NB):
                        plsc.addupdate(tok_v.at[p, b * CS + r, sl], pv)
                return carry

            lax.fori_loop(0, CS, row_body, 0)

            for b in range(NB):
                pltpu.async_copy(tok_v.at[p, pl.ds(b * CS, CS)],
                                 out_hbm.at[pl.ds(b * S + s0 + g * CS, CS)],
                                 ssem[p])

        def ring_body(gq, carry):
            for q in range(NBUF):
                section(NBUF * gq + q, q)
            return carry

        lax.fori_loop(0, NG // NBUF, ring_body, 0)

        for p in range(NBUF):
            drain_stores(p)

    out = emb_kernel(idx_flat, token_table, pos_table)
    return out.reshape(B, S, D)


# speedup vs baseline: 1.1040x; 1.0088x over previous
import functools

import jax
import jax.numpy as jnp
from jax import lax
from jax.experimental import pallas as pl
from jax.experimental.pallas import tpu as pltpu
from jax.experimental.pallas import tpu_sc as plsc

L = 16
NB = 4
CS = 8


def kernel(input_tensor, token_table, pos_table):
    B, S = input_tensor.shape
    V, D = token_table.shape
    N = B * S
    NC, NS = 2, 16
    NW = NC * NS
    SPW = S // NW
    NG = SPW // CS
    CR = NB * CS

    idx_flat = input_tensor.reshape(N).astype(jnp.int32)

    mesh = plsc.VectorSubcoreMesh(core_axis_name="c", subcore_axis_name="s")

    @functools.partial(
        pl.kernel,
        mesh=mesh,
        out_type=jax.ShapeDtypeStruct((N, D), jnp.float32),
        scratch_types=[
            pltpu.VMEM((NB, SPW), jnp.int32),
            pltpu.VMEM((NG, CR), jnp.int32),
            pltpu.VMEM((2, CR, D), jnp.float32),
            pltpu.VMEM((2, CS, D), jnp.float32),
            pltpu.SemaphoreType.DMA,
            pltpu.SemaphoreType.DMA,
            pltpu.SemaphoreType.DMA,
            pltpu.SemaphoreType.DMA,
            pltpu.SemaphoreType.DMA,
            pltpu.SemaphoreType.DMA,
        ],
    )
    def emb_kernel(idx_hbm, tok_hbm, pos_hbm, out_hbm,
                   idx_va, idx2, tok_v, pos_v,
                   gsem0, gsem1, psem0, psem1, ssem0, ssem1):
        gsem = (gsem0, gsem1)
        psem = (psem0, psem1)
        ssem = (ssem0, ssem1)
        wid = lax.axis_index("s") * NC + lax.axis_index("c")
        s0 = wid * SPW

        for b in range(NB):
            pltpu.sync_copy(idx_hbm.at[pl.ds(b * S + s0, SPW)], idx_va.at[b])
        for g in range(NG):
            for b in range(NB):
                idx2[g, pl.ds(b * CS, CS)] = idx_va[b, pl.ds(g * CS, CS)]

        def issue(g, p):
            pltpu.async_copy(pos_hbm.at[pl.ds(s0 + g * CS, CS)],
                             pos_v.at[p], psem[p])
            pltpu.async_copy(tok_hbm.at[idx2.at[g]], tok_v.at[p], gsem[p])

        def drain_stores(p):
            for b in range(NB):
                pltpu.make_async_copy(tok_v.at[p, pl.ds(b * CS, CS)],
                                      out_hbm.at[pl.ds(0, CS)],
                                      ssem[p]).wait()

        issue(0, 0)

        def section(g, p):
            pn = 1 - p

            @pl.when(g + 1 < NG)
            def _():
                @pl.when(g >= 1)
                def _():
                    drain_stores(pn)
                issue(g + 1, pn)

            pltpu.make_async_copy(tok_hbm.at[idx2.at[0]],
                                  tok_v.at[p], gsem[p]).wait()
            pltpu.make_async_copy(pos_hbm.at[pl.ds(0, CS)],
                                  pos_v.at[p], psem[p]).wait()

            def row_body(r, carry):
                for d in range(D // L):
                    sl = pl.ds(d * L, L)
                    pv = pos_v[p, r, sl]
                    for b in range(NB):
                        plsc.addupdate(tok_v.at[p, b * CS + r, sl], pv)
                return carry

            lax.fori_loop(0, CS, row_body, 0)

            for b in range(NB):
                pltpu.async_copy(tok_v.at[p, pl.ds(b * CS, CS)],
                                 out_hbm.at[pl.ds(b * S + s0 + g * CS, CS)],
                                 ssem[p])

        def pair_body(gp, carry):
            section(2 * gp, 0)
            section(2 * gp + 1, 1)
            return carry

        lax.fori_loop(0, NG // 2, pair_body, 0)

        drain_stores(0)
        drain_stores(1)

    out = emb_kernel(idx_flat, token_table, pos_table)
    return out.reshape(B, S, D)


# speedup vs baseline: 1.1283x; 1.0220x over previous
import functools

import jax
import jax.numpy as jnp
from jax import lax
from jax.experimental import pallas as pl
from jax.experimental.pallas import tpu as pltpu
from jax.experimental.pallas import tpu_sc as plsc

L = 16
NB = 4
CS = 8


def kernel(input_tensor, token_table, pos_table):
    B, S = input_tensor.shape
    V, D = token_table.shape
    N = B * S
    NC, NS = 2, 16
    NW = NC * NS
    SPW = S // NW
    NG = SPW // CS
    CR = NB * CS

    idx_flat = input_tensor.reshape(N).astype(jnp.int32)

    mesh = plsc.VectorSubcoreMesh(core_axis_name="c", subcore_axis_name="s")

    @functools.partial(
        pl.kernel,
        mesh=mesh,
        out_type=jax.ShapeDtypeStruct((N, D), jnp.float32),
        scratch_types=[
            pltpu.VMEM((NB, SPW), jnp.int32),
            pltpu.VMEM((NG, CR), jnp.int32),
            pltpu.VMEM((2, CR, D), jnp.float32),
            pltpu.VMEM((2, CS, D), jnp.float32),
            pltpu.SemaphoreType.DMA,
            pltpu.SemaphoreType.DMA,
            pltpu.SemaphoreType.DMA,
            pltpu.SemaphoreType.DMA,
            pltpu.SemaphoreType.DMA,
            pltpu.SemaphoreType.DMA,
        ],
    )
    def emb_kernel(idx_hbm, tok_hbm, pos_hbm, out_hbm,
                   idx_va, idx2, tok_v, pos_v,
                   gsem0, gsem1, psem0, psem1, ssem0, ssem1):
        gsem = (gsem0, gsem1)
        psem = (psem0, psem1)
        ssem = (ssem0, ssem1)
        wid = lax.axis_index("s") * NC + lax.axis_index("c")
        s0 = wid * SPW

        pltpu.async_copy(pos_hbm.at[pl.ds(s0, CS)], pos_v.at[0], psem0)

        for b in range(NB):
            pltpu.async_copy(idx_hbm.at[pl.ds(b * S + s0, SPW)],
                             idx_va.at[b], gsem0)
        for b in range(NB):
            pltpu.make_async_copy(idx_hbm.at[pl.ds(0, SPW)],
                                  idx_va.at[b], gsem0).wait()
        for b in range(NB):
            idx2[0, pl.ds(b * CS, CS)] = idx_va[b, pl.ds(0, CS)]
        pltpu.async_copy(tok_hbm.at[idx2.at[0]], tok_v.at[0], gsem0)
        for g in range(1, NG):
            for b in range(NB):
                idx2[g, pl.ds(b * CS, CS)] = idx_va[b, pl.ds(g * CS, CS)]

        def issue(g, p):
            pltpu.async_copy(pos_hbm.at[pl.ds(s0 + g * CS, CS)],
                             pos_v.at[p], psem[p])
            pltpu.async_copy(tok_hbm.at[idx2.at[g]], tok_v.at[p], gsem[p])

        def drain_stores(p):
            for b in range(NB):
                pltpu.make_async_copy(tok_v.at[p, pl.ds(b * CS, CS)],
                                      out_hbm.at[pl.ds(0, CS)],
                                      ssem[p]).wait()


        def section(g, p):
            pn = 1 - p

            @pl.when(g + 1 < NG)
            def _():
                @pl.when(g >= 1)
                def _():
                    drain_stores(pn)
                issue(g + 1, pn)

            pltpu.make_async_copy(tok_hbm.at[idx2.at[0]],
                                  tok_v.at[p], gsem[p]).wait()
            pltpu.make_async_copy(pos_hbm.at[pl.ds(0, CS)],
                                  pos_v.at[p], psem[p]).wait()

            def row_body(r, carry):
                for d in range(D // L):
                    sl = pl.ds(d * L, L)
                    pv = pos_v[p, r, sl]
                    for b in range(NB):
                        plsc.addupdate(tok_v.at[p, b * CS + r, sl], pv)
                return carry

            lax.fori_loop(0, CS, row_body, 0)

            for b in range(NB):
                pltpu.async_copy(tok_v.at[p, pl.ds(b * CS, CS)],
                                 out_hbm.at[pl.ds(b * S + s0 + g * CS, CS)],
                                 ssem[p])

        def pair_body(gp, carry):
            section(2 * gp, 0)
            section(2 * gp + 1, 1)
            return carry

        lax.fori_loop(0, NG // 2, pair_body, 0)

        drain_stores(0)
        drain_stores(1)

    out = emb_kernel(idx_flat, token_table, pos_table)
    return out.reshape(B, S, D)
